# Initial kernel scaffold; baseline (speedup 1.0000x reference)
import jax
import jax.numpy as jnp
from jax.experimental import pallas as pl


def kernel(voxel_features, nbr_idx1, nbr_idx2, nbr_idx3, nbr_idx4, idx_d2, idx_d3, idx_d4, idx_out, coords_out, params):
    raise NotImplementedError("write your pallas kernel here")


# speedup vs baseline: 8.9326x; 8.9326x over previous
import functools

import jax
import jax.numpy as jnp
from jax import lax
from jax.experimental import pallas as pl
from jax.experimental.pallas import tpu as pltpu
from jax.experimental.pallas import tpu_sc as plsc

_N1, _N2, _N3, _N4, _N5 = 50000, 25000, 12000, 6000, 3000
_P1, _P2, _P3, _P4, _P5 = 50176, 25088, 12288, 6144, 3072
_GW = 128
_GRID_HW = 35200
_BW = 3200


def _sc_gather(table, idx_flat):
    M = idx_flat.shape[1]
    C = table.shape[1]
    mesh = plsc.VectorSubcoreMesh(core_axis_name="core", subcore_axis_name="subcore")

    @functools.partial(
        pl.kernel,
        out_type=jax.ShapeDtypeStruct((M, C), jnp.float32),
        mesh=mesh,
        compiler_params=pltpu.CompilerParams(use_tc_tiling_on_sc=False),
    )
    def gk(x_hbm, i_hbm, o_hbm):
        def body(i_vmem, o_vmem):
            pltpu.sync_copy(x_hbm.at[i_vmem.at[0]], o_vmem)

        pltpu.emit_pipeline(
            body,
            grid=(M // _GW,),
            in_specs=[pl.BlockSpec((1, _GW), index_map=lambda i: (0, i))],
            out_specs=[pl.BlockSpec((_GW, C), index_map=lambda i: (i, 0))],
            core_axis_name=("core", "subcore"),
            dimension_semantics=(pltpu.PARALLEL,),
        )(i_hbm, o_hbm)

    return gk(table, idx_flat)


def _tc_mm(g, wf, s, b, bn=512):
    npad, kc = g.shape
    cout = wf.shape[1]

    def body(g_ref, w_ref, s_ref, b_ref, y_ref):
        y = jnp.dot(g_ref[...], w_ref[...], preferred_element_type=jnp.float32)
        y_ref[...] = jnp.maximum(y * s_ref[...] + b_ref[...], 0.0)

    return pl.pallas_call(
        body,
        grid=(npad // bn,),
        in_specs=[
            pl.BlockSpec((bn, kc), lambda i: (i, 0)),
            pl.BlockSpec((kc, cout), lambda i: (0, 0)),
            pl.BlockSpec((1, cout), lambda i: (0, 0)),
            pl.BlockSpec((1, cout), lambda i: (0, 0)),
        ],
        out_specs=pl.BlockSpec((bn, cout), lambda i: (i, 0)),
        out_shape=jax.ShapeDtypeStruct((npad, cout), jnp.float32),
    )(g, wf, s.reshape(1, cout), b.reshape(1, cout))


def _tc_seg(x1, wseg, bn=512):
    npad = x1.shape[0]

    def body(x_ref, w_ref, p_ref, sc_ref):
        p = jnp.dot(x_ref[...], w_ref[...], preferred_element_type=jnp.float32)
        p_ref[...] = p
        sc_ref[...] = jax.nn.sigmoid(p)

    return pl.pallas_call(
        body,
        grid=(npad // bn,),
        in_specs=[
            pl.BlockSpec((bn, 16), lambda i: (i, 0)),
            pl.BlockSpec((16, 1), lambda i: (0, 0)),
        ],
        out_specs=[
            pl.BlockSpec((bn, 1), lambda i: (i, 0)),
            pl.BlockSpec((bn, 1), lambda i: (i, 0)),
        ],
        out_shape=[
            jax.ShapeDtypeStruct((npad, 1), jnp.float32),
            jax.ShapeDtypeStruct((npad, 1), jnp.float32),
        ],
    )(x1, wseg)


def _tc_scatter_transpose(vals, coords):
    nv = coords.shape[0]

    def body(coords_ref, vals_ref, out_ref, scratch):
        j = pl.program_id(0)

        @pl.when(j == 0)
        def _():
            scratch[...] = jnp.zeros_like(scratch)

            def sc(n, carry):
                c = coords_ref[n]
                scratch[pl.ds(c, 1), :] = vals_ref[pl.ds(n, 1), :]
                return carry

            lax.fori_loop(0, nv, sc, 0)

        for d in range(2):
            blk = scratch[pl.ds(d * _GRID_HW + j * _BW, _BW), :]
            out_ref[:, d, :] = blk.T

    return pl.pallas_call(
        body,
        grid=(_GRID_HW // _BW,),
        in_specs=[
            pl.BlockSpec(memory_space=pltpu.SMEM),
            pl.BlockSpec((vals.shape[0], 128), lambda j: (0, 0)),
        ],
        out_specs=pl.BlockSpec((128, 2, _BW), lambda j: (0, 0, j)),
        out_shape=jax.ShapeDtypeStruct((128, 2, _GRID_HW), jnp.float32),
        scratch_shapes=[pltpu.VMEM((2 * _GRID_HW, 128), jnp.float32)],
    )(coords, vals)


def _pad_rows(a, p):
    return jnp.pad(a, ((0, p - a.shape[0]), (0, 0)))


def _flat_idx(idx, p):
    n, k = idx.shape
    return jnp.pad(idx, ((0, p - n), (0, 0))).reshape(1, p * k)


def kernel(voxel_features, nbr_idx1, nbr_idx2, nbr_idx3, nbr_idx4,
           idx_d2, idx_d3, idx_d4, idx_out, coords_out, params):
    p = params
    f1 = _flat_idx(nbr_idx1, _P1)
    fd2 = _flat_idx(idx_d2, _P2)
    f2 = _flat_idx(nbr_idx2, _P2)
    fd3 = _flat_idx(idx_d3, _P3)
    f3 = _flat_idx(nbr_idx3, _P3)
    fd4 = _flat_idx(idx_d4, _P4)
    f4 = _flat_idx(nbr_idx4, _P4)
    fo = _flat_idx(idx_out, _P5)
    xv = _pad_rows(voxel_features, _P1)

    def block(x, fidx, p_out, name, k=27):
        w = p[name + "_W"]
        cin, cout = w.shape[1], w.shape[2]
        if cin < 8:
            w = jnp.pad(w, ((0, 0), (0, 8 - cin), (0, 0)))
            cin = 8
        g = _sc_gather(x, fidx).reshape(p_out, k * cin)
        return _tc_mm(g, w.reshape(k * cin, cout), p[name + "_s"], p[name + "_b"])

    xv = jnp.pad(xv, ((0, 0), (0, 4)))
    x0 = block(xv, f1, _P1, "c0")
    x1 = block(x0, f1, _P1, "c1")
    x2 = block(x1, fd2, _P2, "c2a")
    x2 = block(x2, f2, _P2, "c2b")
    x2 = block(x2, f2, _P2, "c2c")
    x3 = block(x2, fd3, _P3, "c3a")
    x3 = block(x3, f3, _P3, "c3b")
    x3 = block(x3, f3, _P3, "c3c")
    x4 = block(x3, fd4, _P4, "c4a")
    x4 = block(x4, f4, _P4, "c4b")
    x4 = block(x4, f4, _P4, "c4c")
    out = block(x4, fo, _P5, "cout", k=3)

    spatial = _tc_scatter_transpose(out, coords_out).reshape(1, 256, 200, 176)
    seg_pred, seg_score = _tc_seg(x1, p["Wseg"])
    return spatial, seg_pred[:_N1], seg_score[:_N1]
